# baseline (speedup 1.0000x reference)
import dataclasses

import jax
import jax.numpy as jnp
from jax import lax
from jax.experimental import pallas as pl
from jax.experimental.pallas import tpu as pltpu
from jax.experimental.pallas import tpu_sc as plsc

B = 16384
V_SIZE = 1000000
D = 64
C_POS = 20
C_NEG = 50
C = C_POS + C_NEG
C_PAD = 80
NW = 32
BPW = B // NW
NB = 8
STEPS = BPW // NB
ROWS_STEP = NB * C
GCHUNK = 112
NGC = ROWS_STEP // GCHUNK


def _transpose_body(t_ref, o_ref):
    o_ref[...] = t_ref[...].T


@jax.jit
def _tc_transpose(table_t):
    blk = 2048
    grid = (V_SIZE + blk - 1) // blk
    return pl.pallas_call(
        _transpose_body,
        grid=(grid,),
        in_specs=[pl.BlockSpec((D, blk), lambda i: (0, i))],
        out_specs=pl.BlockSpec((blk, D), lambda i: (i, 0)),
        out_shape=jax.ShapeDtypeStruct((V_SIZE, D), jnp.float32),
    )(table_t)


def _sc_body(u_hbm, v_hbm, idx_hbm, x_hbm, out_hbm,
             xbuf, vcs, idx_v, rows, out_v, sem):
    wid = lax.axis_index("s") * 2 + lax.axis_index("c")
    base = wid * BPW

    lane = lax.iota(jnp.int32, 16)
    masks = [lane == j for j in range(16)]

    pltpu.sync_copy(x_hbm.at[pl.ds(base, BPW)], xbuf)
    vc_copies = [
        pltpu.async_copy(
            v_hbm.at[xbuf.at[pl.ds(k * 128, 128)]],
            vcs.at[pl.ds(k * 128, 128)], sem)
        for k in range(BPW // 128)
    ]
    for cp in vc_copies:
        cp.wait()

    @pl.loop(0, STEPS)
    def _step(s):
        b0 = base + s * NB
        pltpu.sync_copy(idx_hbm.at[pl.ds(b0 * C, ROWS_STEP)], idx_v)
        u_copies = [
            pltpu.async_copy(
                u_hbm.at[idx_v.at[pl.ds(k * GCHUNK, GCHUNK)]],
                rows.at[pl.ds(k * GCHUNK, GCHUNK)], sem)
            for k in range(NGC)
        ]
        for cp in u_copies:
            cp.wait()

        @pl.loop(0, NB)
        def _row(i):
            bb = s * NB + i
            vc0 = vcs[bb, pl.ds(0, 16)]
            vc1 = vcs[bb, pl.ds(16, 16)]
            vc2 = vcs[bb, pl.ds(32, 16)]
            vc3 = vcs[bb, pl.ds(48, 16)]
            accs = [jnp.zeros((16,), jnp.float32) for _ in range(5)]
            for j in range(C):
                r = i * C + j
                t = rows[r, pl.ds(0, 16)] * vc0
                t = t + rows[r, pl.ds(16, 16)] * vc1
                t = t + rows[r, pl.ds(32, 16)] * vc2
                t = t + rows[r, pl.ds(48, 16)] * vc3
                sv = jnp.sum(t)
                g, l = divmod(j, 16)
                accs[g] = jnp.where(masks[l], sv, accs[g])
            for g in range(5):
                out_v[i, pl.ds(g * 16, 16)] = accs[g]

        pltpu.sync_copy(out_v, out_hbm.at[pl.ds(b0, NB)])


@jax.jit
def _sc_dots(u_weight, v_weight, idx_all, x):
    mesh = plsc.VectorSubcoreMesh(core_axis_name="c", subcore_axis_name="s")
    cp = pltpu.CompilerParams()
    if "needs_layout_passes" in pltpu.CompilerParams.__dataclass_fields__:
        cp = dataclasses.replace(cp, needs_layout_passes=False)
    if "use_tc_tiling_on_sc" in pltpu.CompilerParams.__dataclass_fields__:
        cp = dataclasses.replace(cp, use_tc_tiling_on_sc=False)
    kern = pl.kernel(
        _sc_body,
        out_type=jax.ShapeDtypeStruct((B, C_PAD), jnp.float32),
        mesh=mesh,
        scratch_types=[
            pltpu.VMEM((BPW,), jnp.int32),
            pltpu.VMEM((BPW, D), jnp.float32),
            pltpu.VMEM((ROWS_STEP,), jnp.int32),
            pltpu.VMEM((ROWS_STEP, D), jnp.float32),
            pltpu.VMEM((NB, C_PAD), jnp.float32),
            pltpu.SemaphoreType.DMA,
        ],
        compiler_params=cp,
    )
    return kern(u_weight, v_weight, idx_all, x)


def _tc_body(uv_ref, o_ref):
    z = uv_ref[...]
    col = lax.broadcasted_iota(jnp.int32, z.shape, 1)
    pos = col < C_POS
    valid = col < C
    zs = jnp.where(pos, z, -z)
    ls = jnp.minimum(zs, 0.0) - jnp.log1p(jnp.exp(-jnp.abs(zs)))
    contrib = jnp.where(valid, ls, 0.0)
    o_ref[...] = -jnp.sum(contrib, axis=1)


@jax.jit
def _tc_epilogue(uv):
    blk = 2048
    return pl.pallas_call(
        _tc_body,
        grid=(B // blk,),
        in_specs=[pl.BlockSpec((blk, C_PAD), lambda i: (i, 0))],
        out_specs=pl.BlockSpec((blk,), lambda i: (i,)),
        out_shape=jax.ShapeDtypeStruct((B,), jnp.float32),
    )(uv)


def kernel(x, positive_w, negative_w, V_weight, U_weight):
    idx_all = jnp.concatenate(
        [positive_w.astype(jnp.int32), negative_w.astype(jnp.int32)], axis=1
    ).reshape(-1)
    u_rm = _tc_transpose(U_weight.T)
    v_rm = _tc_transpose(V_weight.T)
    uv = _sc_dots(u_rm, v_rm, idx_all, x.astype(jnp.int32))
    return _tc_epilogue(uv)


# speedup vs baseline: 2.3323x; 2.3323x over previous
import dataclasses

import jax
import jax.numpy as jnp
from jax import lax
from jax.experimental import pallas as pl
from jax.experimental.pallas import tpu as pltpu
from jax.experimental.pallas import tpu_sc as plsc

B = 16384
V_SIZE = 1000000
D = 64
C_POS = 20
C_NEG = 50
C = C_POS + C_NEG
C_PAD = 80
NW = 32
BPW = B // NW
NB = 8
STEPS = BPW // NB
ROWS_STEP = NB * C
GCHUNK = 112
NGC = ROWS_STEP // GCHUNK


T_BLK = 4096
SPLIT = T_BLK * 122
OUT_ROWS = V_SIZE - SPLIT
T_GRID = (OUT_ROWS + T_BLK - 1) // T_BLK
TROWS = 2 * OUT_ROWS


def _transpose_body(a_ref, b_ref, o_ref):
    o_ref[:, :D] = a_ref[...].T
    o_ref[:, D:] = b_ref[...].T


@jax.jit
def _tc_transpose(table_t):
    out = pl.pallas_call(
        _transpose_body,
        grid=(T_GRID,),
        in_specs=[
            pl.BlockSpec((D, T_BLK), lambda i: (0, i)),
            pl.BlockSpec((D, T_BLK), lambda i: (0, i + 122)),
        ],
        out_specs=pl.BlockSpec((T_BLK, 2 * D), lambda i: (i, 0)),
        out_shape=jax.ShapeDtypeStruct((OUT_ROWS, 2 * D), jnp.float32),
    )(table_t, table_t)
    return out.reshape(TROWS, D)


def _sc_body(u_hbm, v_hbm, idx_hbm, x_hbm, out_hbm,
             xbuf, vcs, idx_v, rows, out_v, sem):
    wid = lax.axis_index("s") * 2 + lax.axis_index("c")
    base = wid * BPW

    lane = lax.iota(jnp.int32, 16)
    masks = [lane == j for j in range(16)]

    pltpu.sync_copy(x_hbm.at[pl.ds(base, BPW)], xbuf)
    for k in range(BPW // 16):
        xv = xbuf[pl.ds(k * 16, 16)]
        adj = jnp.where(xv >= SPLIT, 2 * SPLIT - 1, 0)
        xbuf[pl.ds(k * 16, 16)] = xv + xv - adj
    vc_copies = [
        pltpu.async_copy(
            v_hbm.at[xbuf.at[pl.ds(k * 128, 128)]],
            vcs.at[pl.ds(k * 128, 128)], sem)
        for k in range(BPW // 128)
    ]
    for cp in vc_copies:
        cp.wait()

    @pl.loop(0, STEPS)
    def _step(s):
        b0 = base + s * NB
        pltpu.sync_copy(idx_hbm.at[pl.ds(b0 * C, ROWS_STEP)], idx_v)
        for k in range(ROWS_STEP // 16):
            iv = idx_v[pl.ds(k * 16, 16)]
            adj = jnp.where(iv >= SPLIT, 2 * SPLIT - 1, 0)
            idx_v[pl.ds(k * 16, 16)] = iv + iv - adj
        u_copies = [
            pltpu.async_copy(
                u_hbm.at[idx_v.at[pl.ds(k * GCHUNK, GCHUNK)]],
                rows.at[pl.ds(k * GCHUNK, GCHUNK)], sem)
            for k in range(NGC)
        ]
        for cp in u_copies:
            cp.wait()

        @pl.loop(0, NB)
        def _row(i):
            bb = s * NB + i
            vc0 = vcs[bb, pl.ds(0, 16)]
            vc1 = vcs[bb, pl.ds(16, 16)]
            vc2 = vcs[bb, pl.ds(32, 16)]
            vc3 = vcs[bb, pl.ds(48, 16)]
            accs = [jnp.zeros((16,), jnp.float32) for _ in range(5)]
            for j in range(C):
                r = i * C + j
                t = rows[r, pl.ds(0, 16)] * vc0
                t = t + rows[r, pl.ds(16, 16)] * vc1
                t = t + rows[r, pl.ds(32, 16)] * vc2
                t = t + rows[r, pl.ds(48, 16)] * vc3
                sv = jnp.sum(t)
                g, l = divmod(j, 16)
                accs[g] = jnp.where(masks[l], sv, accs[g])
            for g in range(5):
                out_v[i, pl.ds(g * 16, 16)] = accs[g]

        pltpu.sync_copy(out_v, out_hbm.at[pl.ds(b0, NB)])


@jax.jit
def _sc_dots(u_weight, v_weight, idx_all, x):
    mesh = plsc.VectorSubcoreMesh(core_axis_name="c", subcore_axis_name="s")
    cp = pltpu.CompilerParams()
    if "needs_layout_passes" in pltpu.CompilerParams.__dataclass_fields__:
        cp = dataclasses.replace(cp, needs_layout_passes=False)
    if "use_tc_tiling_on_sc" in pltpu.CompilerParams.__dataclass_fields__:
        cp = dataclasses.replace(cp, use_tc_tiling_on_sc=False)
    kern = pl.kernel(
        _sc_body,
        out_type=jax.ShapeDtypeStruct((B, C_PAD), jnp.float32),
        mesh=mesh,
        scratch_types=[
            pltpu.VMEM((BPW,), jnp.int32),
            pltpu.VMEM((BPW, D), jnp.float32),
            pltpu.VMEM((ROWS_STEP,), jnp.int32),
            pltpu.VMEM((ROWS_STEP, D), jnp.float32),
            pltpu.VMEM((NB, C_PAD), jnp.float32),
            pltpu.SemaphoreType.DMA,
        ],
        compiler_params=cp,
    )
    return kern(u_weight, v_weight, idx_all, x)


def _tc_body(uv_ref, o_ref):
    z = uv_ref[...]
    col = lax.broadcasted_iota(jnp.int32, z.shape, 1)
    pos = col < C_POS
    valid = col < C
    zs = jnp.where(pos, z, -z)
    ls = jnp.minimum(zs, 0.0) - jnp.log1p(jnp.exp(-jnp.abs(zs)))
    contrib = jnp.where(valid, ls, 0.0)
    o_ref[...] = -jnp.sum(contrib, axis=1)


@jax.jit
def _tc_epilogue(uv):
    blk = 2048
    return pl.pallas_call(
        _tc_body,
        grid=(B // blk,),
        in_specs=[pl.BlockSpec((blk, C_PAD), lambda i: (i, 0))],
        out_specs=pl.BlockSpec((blk,), lambda i: (i,)),
        out_shape=jax.ShapeDtypeStruct((B,), jnp.float32),
    )(uv)


def kernel(x, positive_w, negative_w, V_weight, U_weight):
    idx_all = jnp.concatenate(
        [positive_w.astype(jnp.int32), negative_w.astype(jnp.int32)], axis=1
    ).reshape(-1)
    u_rm = _tc_transpose(U_weight.T)
    v_rm = _tc_transpose(V_weight.T)
    uv = _sc_dots(u_rm, v_rm, idx_all, x.astype(jnp.int32))
    return _tc_epilogue(uv)


# speedup vs baseline: 2.7703x; 1.1878x over previous
import dataclasses

import jax
import jax.numpy as jnp
from jax import lax
from jax.experimental import pallas as pl
from jax.experimental.pallas import tpu as pltpu
from jax.experimental.pallas import tpu_sc as plsc

B = 16384
V_SIZE = 1000000
D = 64
C_POS = 20
C_NEG = 50
C = C_POS + C_NEG
C_PAD = 80
NW = 32
BPW = B // NW
NB = 8
STEPS = BPW // NB
ROWS_STEP = NB * C
GCHUNK = 112
NGC = ROWS_STEP // GCHUNK


T_BLK = 4096
SPLIT = T_BLK * 122
OUT_ROWS = V_SIZE - SPLIT
T_GRID = (OUT_ROWS + T_BLK - 1) // T_BLK
TROWS = 2 * OUT_ROWS


def _transpose_body(a_ref, b_ref, o_ref):
    o_ref[:, :D] = a_ref[...].T
    o_ref[:, D:] = b_ref[...].T


@jax.jit
def _tc_transpose(table_t):
    out = pl.pallas_call(
        _transpose_body,
        grid=(T_GRID,),
        in_specs=[
            pl.BlockSpec((D, T_BLK), lambda i: (0, i)),
            pl.BlockSpec((D, T_BLK), lambda i: (0, i + 122)),
        ],
        out_specs=pl.BlockSpec((T_BLK, 2 * D), lambda i: (i, 0)),
        out_shape=jax.ShapeDtypeStruct((OUT_ROWS, 2 * D), jnp.float32),
    )(table_t, table_t)
    return out.reshape(TROWS, D)


def _sc_body(u_hbm, v_hbm, idx_hbm, x_hbm, out_hbm,
             xbuf, vcs, idx_a, idx_b, rows_a, rows_b, out_a, out_b,
             sem, gsem_a, gsem_b, osem_a, osem_b):
    wid = lax.axis_index("s") * 2 + lax.axis_index("c")
    base = wid * BPW

    lane = lax.iota(jnp.int32, 16)
    masks = [lane == j for j in range(16)]

    def fire_step(s, idx_v, rows, gsem):
        b0 = base + s * NB
        pltpu.sync_copy(idx_hbm.at[pl.ds(b0 * C, ROWS_STEP)], idx_v)
        for k in range(ROWS_STEP // 16):
            iv = idx_v[pl.ds(k * 16, 16)]
            adj = jnp.where(iv >= SPLIT, 2 * SPLIT - 1, 0)
            idx_v[pl.ds(k * 16, 16)] = iv + iv - adj
        for k in range(NGC):
            pltpu.async_copy(
                u_hbm.at[idx_v.at[pl.ds(k * GCHUNK, GCHUNK)]],
                rows.at[pl.ds(k * GCHUNK, GCHUNK)], gsem)

    def wait_step(idx_v, rows, gsem):
        for k in range(NGC):
            pltpu.make_async_copy(
                u_hbm.at[idx_v.at[pl.ds(k * GCHUNK, GCHUNK)]],
                rows.at[pl.ds(k * GCHUNK, GCHUNK)], gsem).wait()

    def wait_out(out_v, osem):
        pltpu.make_async_copy(out_v, out_hbm.at[pl.ds(base, NB)],
                              osem).wait()

    def compute_step(s, rows, out_v, osem):
        @pl.loop(0, NB)
        def _row(i):
            bb = s * NB + i
            vc0 = vcs[bb, pl.ds(0, 16)]
            vc1 = vcs[bb, pl.ds(16, 16)]
            vc2 = vcs[bb, pl.ds(32, 16)]
            vc3 = vcs[bb, pl.ds(48, 16)]
            accs = [jnp.zeros((16,), jnp.float32) for _ in range(5)]
            for j in range(C):
                r = i * C + j
                t = rows[r, pl.ds(0, 16)] * vc0
                t = t + rows[r, pl.ds(16, 16)] * vc1
                t = t + rows[r, pl.ds(32, 16)] * vc2
                t = t + rows[r, pl.ds(48, 16)] * vc3
                sv = jnp.sum(t)
                g, l = divmod(j, 16)
                accs[g] = jnp.where(masks[l], sv, accs[g])
            for g in range(5):
                out_v[i, pl.ds(g * 16, 16)] = accs[g]

        pltpu.async_copy(out_v, out_hbm.at[pl.ds(base + s * NB, NB)], osem)

    pltpu.sync_copy(x_hbm.at[pl.ds(base, BPW)], xbuf)
    for k in range(BPW // 16):
        xv = xbuf[pl.ds(k * 16, 16)]
        adj = jnp.where(xv >= SPLIT, 2 * SPLIT - 1, 0)
        xbuf[pl.ds(k * 16, 16)] = xv + xv - adj
    vc_copies = [
        pltpu.async_copy(
            v_hbm.at[xbuf.at[pl.ds(k * 128, 128)]],
            vcs.at[pl.ds(k * 128, 128)], sem)
        for k in range(BPW // 128)
    ]
    for cp in vc_copies:
        cp.wait()

    fire_step(0, idx_a, rows_a, gsem_a)

    @pl.loop(0, STEPS, step=2)
    def _pair(s):
        fire_step(s + 1, idx_b, rows_b, gsem_b)
        wait_step(idx_a, rows_a, gsem_a)

        @pl.when(s > 0)
        def _():
            wait_out(out_a, osem_a)

        compute_step(s, rows_a, out_a, osem_a)

        @pl.when(s + 2 < STEPS)
        def _():
            fire_step(s + 2, idx_a, rows_a, gsem_a)

        wait_step(idx_b, rows_b, gsem_b)

        @pl.when(s > 0)
        def _():
            wait_out(out_b, osem_b)

        compute_step(s + 1, rows_b, out_b, osem_b)

    wait_out(out_a, osem_a)
    wait_out(out_b, osem_b)


@jax.jit
def _sc_dots(u_weight, v_weight, idx_all, x):
    mesh = plsc.VectorSubcoreMesh(core_axis_name="c", subcore_axis_name="s")
    cp = pltpu.CompilerParams()
    if "needs_layout_passes" in pltpu.CompilerParams.__dataclass_fields__:
        cp = dataclasses.replace(cp, needs_layout_passes=False)
    if "use_tc_tiling_on_sc" in pltpu.CompilerParams.__dataclass_fields__:
        cp = dataclasses.replace(cp, use_tc_tiling_on_sc=False)
    kern = pl.kernel(
        _sc_body,
        out_type=jax.ShapeDtypeStruct((B, C_PAD), jnp.float32),
        mesh=mesh,
        scratch_types=[
            pltpu.VMEM((BPW,), jnp.int32),
            pltpu.VMEM((BPW, D), jnp.float32),
            pltpu.VMEM((ROWS_STEP,), jnp.int32),
            pltpu.VMEM((ROWS_STEP,), jnp.int32),
            pltpu.VMEM((ROWS_STEP, D), jnp.float32),
            pltpu.VMEM((ROWS_STEP, D), jnp.float32),
            pltpu.VMEM((NB, C_PAD), jnp.float32),
            pltpu.VMEM((NB, C_PAD), jnp.float32),
            pltpu.SemaphoreType.DMA,
            pltpu.SemaphoreType.DMA,
            pltpu.SemaphoreType.DMA,
            pltpu.SemaphoreType.DMA,
            pltpu.SemaphoreType.DMA,
        ],
        compiler_params=cp,
    )
    return kern(u_weight, v_weight, idx_all, x)


def _tc_body(uv_ref, o_ref):
    z = uv_ref[...]
    col = lax.broadcasted_iota(jnp.int32, z.shape, 1)
    pos = col < C_POS
    valid = col < C
    zs = jnp.where(pos, z, -z)
    ls = jnp.minimum(zs, 0.0) - jnp.log1p(jnp.exp(-jnp.abs(zs)))
    contrib = jnp.where(valid, ls, 0.0)
    o_ref[...] = -jnp.sum(contrib, axis=1)


@jax.jit
def _tc_epilogue(uv):
    blk = 2048
    return pl.pallas_call(
        _tc_body,
        grid=(B // blk,),
        in_specs=[pl.BlockSpec((blk, C_PAD), lambda i: (i, 0))],
        out_specs=pl.BlockSpec((blk,), lambda i: (i,)),
        out_shape=jax.ShapeDtypeStruct((B,), jnp.float32),
    )(uv)


def kernel(x, positive_w, negative_w, V_weight, U_weight):
    idx_all = jnp.concatenate(
        [positive_w.astype(jnp.int32), negative_w.astype(jnp.int32)], axis=1
    ).reshape(-1)
    u_rm = _tc_transpose(U_weight.T)
    v_rm = _tc_transpose(V_weight.T)
    uv = _sc_dots(u_rm, v_rm, idx_all, x.astype(jnp.int32))
    return _tc_epilogue(uv)


# speedup vs baseline: 3.2683x; 1.1797x over previous
import dataclasses

import jax
import jax.numpy as jnp
from jax import lax
from jax.experimental import pallas as pl
from jax.experimental.pallas import tpu as pltpu
from jax.experimental.pallas import tpu_sc as plsc

B = 16384
V_SIZE = 1000000
D = 64
C_POS = 20
C_NEG = 50
C = C_POS + C_NEG
C_PAD = 80
NW = 32
BPW = B // NW
NB = 8
STEPS = BPW // NB
ROWS_STEP = NB * C
GCHUNK = 112
NGC = ROWS_STEP // GCHUNK


T_BLK = 4096
T_GRID = (V_SIZE + T_BLK - 1) // T_BLK
TROWS = 2 * V_SIZE


def _transpose_body(a_ref, b_ref, o_ref):
    o_ref[...] = jnp.concatenate([a_ref[...], b_ref[...]], axis=0).T


@jax.jit
def _tc_transpose(u_t, v_t):
    out = pl.pallas_call(
        _transpose_body,
        grid=(T_GRID,),
        in_specs=[
            pl.BlockSpec((D, T_BLK), lambda i: (0, i)),
            pl.BlockSpec((D, T_BLK), lambda i: (0, i)),
        ],
        out_specs=pl.BlockSpec((T_BLK, 2 * D), lambda i: (i, 0)),
        out_shape=jax.ShapeDtypeStruct((V_SIZE, 2 * D), jnp.float32),
    )(u_t, v_t)
    return out.reshape(TROWS, D)


def _sc_body(w_hbm, idx_hbm, x_hbm, out_hbm,
             xbuf, vcs, idx_a, idx_b, rows_a, rows_b, out_a, out_b,
             sem, gsem_a, gsem_b, osem_a, osem_b):
    wid = lax.axis_index("s") * 2 + lax.axis_index("c")
    base = wid * BPW

    lane = lax.iota(jnp.int32, 16)
    masks = [lane == j for j in range(16)]

    def fire_step(s, idx_v, rows, gsem):
        b0 = base + s * NB
        pltpu.sync_copy(idx_hbm.at[pl.ds(b0 * C, ROWS_STEP)], idx_v)
        for k in range(ROWS_STEP // 16):
            iv = idx_v[pl.ds(k * 16, 16)]
            idx_v[pl.ds(k * 16, 16)] = iv + iv
        for k in range(NGC):
            pltpu.async_copy(
                w_hbm.at[idx_v.at[pl.ds(k * GCHUNK, GCHUNK)]],
                rows.at[pl.ds(k * GCHUNK, GCHUNK)], gsem)

    def wait_step(idx_v, rows, gsem):
        for k in range(NGC):
            pltpu.make_async_copy(
                w_hbm.at[idx_v.at[pl.ds(k * GCHUNK, GCHUNK)]],
                rows.at[pl.ds(k * GCHUNK, GCHUNK)], gsem).wait()

    def wait_out(out_v, osem):
        pltpu.make_async_copy(out_v, out_hbm.at[pl.ds(base, NB)],
                              osem).wait()

    def compute_step(s, rows, out_v, osem):
        @pl.loop(0, NB)
        def _row(i):
            bb = s * NB + i
            vc0 = vcs[bb, pl.ds(0, 16)]
            vc1 = vcs[bb, pl.ds(16, 16)]
            vc2 = vcs[bb, pl.ds(32, 16)]
            vc3 = vcs[bb, pl.ds(48, 16)]
            accs = [jnp.zeros((16,), jnp.float32) for _ in range(5)]
            for j in range(C):
                r = i * C + j
                t = rows[r, pl.ds(0, 16)] * vc0
                t = t + rows[r, pl.ds(16, 16)] * vc1
                t = t + rows[r, pl.ds(32, 16)] * vc2
                t = t + rows[r, pl.ds(48, 16)] * vc3
                sv = jnp.sum(t)
                g, l = divmod(j, 16)
                accs[g] = jnp.where(masks[l], sv, accs[g])
            for g in range(5):
                out_v[i, pl.ds(g * 16, 16)] = accs[g]

        pltpu.async_copy(out_v, out_hbm.at[pl.ds(base + s * NB, NB)], osem)

    pltpu.sync_copy(x_hbm.at[pl.ds(base, BPW)], xbuf)
    for k in range(BPW // 16):
        xv = xbuf[pl.ds(k * 16, 16)]
        xbuf[pl.ds(k * 16, 16)] = xv + xv + 1
    vc_copies = [
        pltpu.async_copy(
            w_hbm.at[xbuf.at[pl.ds(k * 128, 128)]],
            vcs.at[pl.ds(k * 128, 128)], sem)
        for k in range(BPW // 128)
    ]
    for cp in vc_copies:
        cp.wait()

    fire_step(0, idx_a, rows_a, gsem_a)

    @pl.loop(0, STEPS, step=2)
    def _pair(s):
        fire_step(s + 1, idx_b, rows_b, gsem_b)
        wait_step(idx_a, rows_a, gsem_a)

        @pl.when(s > 0)
        def _():
            wait_out(out_a, osem_a)

        compute_step(s, rows_a, out_a, osem_a)

        @pl.when(s + 2 < STEPS)
        def _():
            fire_step(s + 2, idx_a, rows_a, gsem_a)

        wait_step(idx_b, rows_b, gsem_b)

        @pl.when(s > 0)
        def _():
            wait_out(out_b, osem_b)

        compute_step(s + 1, rows_b, out_b, osem_b)

    wait_out(out_a, osem_a)
    wait_out(out_b, osem_b)


@jax.jit
def _sc_dots(w_packed, idx_all, x):
    mesh = plsc.VectorSubcoreMesh(core_axis_name="c", subcore_axis_name="s")
    cp = pltpu.CompilerParams()
    if "needs_layout_passes" in pltpu.CompilerParams.__dataclass_fields__:
        cp = dataclasses.replace(cp, needs_layout_passes=False)
    if "use_tc_tiling_on_sc" in pltpu.CompilerParams.__dataclass_fields__:
        cp = dataclasses.replace(cp, use_tc_tiling_on_sc=False)
    kern = pl.kernel(
        _sc_body,
        out_type=jax.ShapeDtypeStruct((B, C_PAD), jnp.float32),
        mesh=mesh,
        scratch_types=[
            pltpu.VMEM((BPW,), jnp.int32),
            pltpu.VMEM((BPW, D), jnp.float32),
            pltpu.VMEM((ROWS_STEP,), jnp.int32),
            pltpu.VMEM((ROWS_STEP,), jnp.int32),
            pltpu.VMEM((ROWS_STEP, D), jnp.float32),
            pltpu.VMEM((ROWS_STEP, D), jnp.float32),
            pltpu.VMEM((NB, C_PAD), jnp.float32),
            pltpu.VMEM((NB, C_PAD), jnp.float32),
            pltpu.SemaphoreType.DMA,
            pltpu.SemaphoreType.DMA,
            pltpu.SemaphoreType.DMA,
            pltpu.SemaphoreType.DMA,
            pltpu.SemaphoreType.DMA,
        ],
        compiler_params=cp,
    )
    return kern(w_packed, idx_all, x)


def _tc_body(uv_ref, o_ref):
    z = uv_ref[...]
    col = lax.broadcasted_iota(jnp.int32, z.shape, 1)
    pos = col < C_POS
    valid = col < C
    zs = jnp.where(pos, z, -z)
    ls = jnp.minimum(zs, 0.0) - jnp.log1p(jnp.exp(-jnp.abs(zs)))
    contrib = jnp.where(valid, ls, 0.0)
    o_ref[...] = -jnp.sum(contrib, axis=1)


@jax.jit
def _tc_epilogue(uv):
    blk = 2048
    return pl.pallas_call(
        _tc_body,
        grid=(B // blk,),
        in_specs=[pl.BlockSpec((blk, C_PAD), lambda i: (i, 0))],
        out_specs=pl.BlockSpec((blk,), lambda i: (i,)),
        out_shape=jax.ShapeDtypeStruct((B,), jnp.float32),
    )(uv)


def kernel(x, positive_w, negative_w, V_weight, U_weight):
    idx_all = jnp.concatenate(
        [positive_w.astype(jnp.int32), negative_w.astype(jnp.int32)], axis=1
    ).reshape(-1)
    w_rm = _tc_transpose(U_weight.T, V_weight.T)
    uv = _sc_dots(w_rm, idx_all, x.astype(jnp.int32))
    return _tc_epilogue(uv)


# speedup vs baseline: 3.5894x; 1.0983x over previous
import dataclasses

import jax
import jax.numpy as jnp
from jax import lax
from jax.experimental import pallas as pl
from jax.experimental.pallas import tpu as pltpu
from jax.experimental.pallas import tpu_sc as plsc

B = 16384
V_SIZE = 1000000
D = 64
C_POS = 20
C_NEG = 50
C = C_POS + C_NEG
C_PAD = 80
NW = 32
BPW = B // NW
NB = 8
STEPS = BPW // NB
ROWS_STEP = NB * C
GCHUNK = 112
NGC = ROWS_STEP // GCHUNK


T_BLK = 8192
T_GRID = (V_SIZE + T_BLK - 1) // T_BLK
TROWS = 2 * V_SIZE


def _transpose_body(a_ref, b_ref, o_ref):
    o_ref[...] = jnp.concatenate([a_ref[...], b_ref[...]], axis=0).T


@jax.jit
def _tc_transpose(u_t, v_t):
    out = pl.pallas_call(
        _transpose_body,
        grid=(T_GRID,),
        in_specs=[
            pl.BlockSpec((D, T_BLK), lambda i: (0, i)),
            pl.BlockSpec((D, T_BLK), lambda i: (0, i)),
        ],
        out_specs=pl.BlockSpec((T_BLK, 2 * D), lambda i: (i, 0)),
        out_shape=jax.ShapeDtypeStruct((V_SIZE, 2 * D), jnp.float32),
    )(u_t, v_t)
    return out.reshape(TROWS, D)


def _sc_body(w_hbm, idx_hbm, x_hbm, out_hbm,
             xbuf, vcs, idx_a, idx_b, rows_a, rows_b, out_a, out_b,
             sem, gsem_a, gsem_b, osem_a, osem_b):
    wid = lax.axis_index("s") * 2 + lax.axis_index("c")
    base = wid * BPW

    lane = lax.iota(jnp.int32, 16)
    masks = [lane == j for j in range(16)]

    def fire_step(s, idx_v, rows, gsem):
        b0 = base + s * NB
        pltpu.sync_copy(idx_hbm.at[pl.ds(b0 * C, ROWS_STEP)], idx_v)
        for k in range(ROWS_STEP // 16):
            iv = idx_v[pl.ds(k * 16, 16)]
            idx_v[pl.ds(k * 16, 16)] = iv + iv
        for k in range(NGC):
            pltpu.async_copy(
                w_hbm.at[idx_v.at[pl.ds(k * GCHUNK, GCHUNK)]],
                rows.at[pl.ds(k * GCHUNK, GCHUNK)], gsem)

    def wait_step(idx_v, rows, gsem):
        for k in range(NGC):
            pltpu.make_async_copy(
                w_hbm.at[idx_v.at[pl.ds(k * GCHUNK, GCHUNK)]],
                rows.at[pl.ds(k * GCHUNK, GCHUNK)], gsem).wait()

    def wait_out(out_v, osem):
        pltpu.make_async_copy(out_v, out_hbm.at[pl.ds(base, NB)],
                              osem).wait()

    def compute_step(s, rows, out_v, osem):
        @pl.loop(0, NB)
        def _row(i):
            bb = s * NB + i
            vc0 = vcs[bb, pl.ds(0, 16)]
            vc1 = vcs[bb, pl.ds(16, 16)]
            vc2 = vcs[bb, pl.ds(32, 16)]
            vc3 = vcs[bb, pl.ds(48, 16)]
            accs = [jnp.zeros((16,), jnp.float32) for _ in range(5)]
            for j in range(C):
                r = i * C + j
                t = rows[r, pl.ds(0, 16)] * vc0
                t = t + rows[r, pl.ds(16, 16)] * vc1
                t = t + rows[r, pl.ds(32, 16)] * vc2
                t = t + rows[r, pl.ds(48, 16)] * vc3
                sv = jnp.sum(t)
                g, l = divmod(j, 16)
                accs[g] = jnp.where(masks[l], sv, accs[g])
            for g in range(5):
                out_v[i, pl.ds(g * 16, 16)] = accs[g]

        pltpu.async_copy(out_v, out_hbm.at[pl.ds(base + s * NB, NB)], osem)

    pltpu.sync_copy(x_hbm.at[pl.ds(base, BPW)], xbuf)
    for k in range(BPW // 16):
        xv = xbuf[pl.ds(k * 16, 16)]
        xbuf[pl.ds(k * 16, 16)] = xv + xv + 1
    vc_copies = [
        pltpu.async_copy(
            w_hbm.at[xbuf.at[pl.ds(k * 128, 128)]],
            vcs.at[pl.ds(k * 128, 128)], sem)
        for k in range(BPW // 128)
    ]
    for cp in vc_copies:
        cp.wait()

    fire_step(0, idx_a, rows_a, gsem_a)

    @pl.loop(0, STEPS, step=2)
    def _pair(s):
        fire_step(s + 1, idx_b, rows_b, gsem_b)
        wait_step(idx_a, rows_a, gsem_a)

        @pl.when(s > 0)
        def _():
            wait_out(out_a, osem_a)

        compute_step(s, rows_a, out_a, osem_a)

        @pl.when(s + 2 < STEPS)
        def _():
            fire_step(s + 2, idx_a, rows_a, gsem_a)

        wait_step(idx_b, rows_b, gsem_b)

        @pl.when(s > 0)
        def _():
            wait_out(out_b, osem_b)

        compute_step(s + 1, rows_b, out_b, osem_b)

    wait_out(out_a, osem_a)
    wait_out(out_b, osem_b)


@jax.jit
def _sc_dots(w_packed, idx_all, x):
    mesh = plsc.VectorSubcoreMesh(core_axis_name="c", subcore_axis_name="s")
    cp = pltpu.CompilerParams()
    if "needs_layout_passes" in pltpu.CompilerParams.__dataclass_fields__:
        cp = dataclasses.replace(cp, needs_layout_passes=False)
    if "use_tc_tiling_on_sc" in pltpu.CompilerParams.__dataclass_fields__:
        cp = dataclasses.replace(cp, use_tc_tiling_on_sc=False)
    kern = pl.kernel(
        _sc_body,
        out_type=jax.ShapeDtypeStruct((B, C_PAD), jnp.float32),
        mesh=mesh,
        scratch_types=[
            pltpu.VMEM((BPW,), jnp.int32),
            pltpu.VMEM((BPW, D), jnp.float32),
            pltpu.VMEM((ROWS_STEP,), jnp.int32),
            pltpu.VMEM((ROWS_STEP,), jnp.int32),
            pltpu.VMEM((ROWS_STEP, D), jnp.float32),
            pltpu.VMEM((ROWS_STEP, D), jnp.float32),
            pltpu.VMEM((NB, C_PAD), jnp.float32),
            pltpu.VMEM((NB, C_PAD), jnp.float32),
            pltpu.SemaphoreType.DMA,
            pltpu.SemaphoreType.DMA,
            pltpu.SemaphoreType.DMA,
            pltpu.SemaphoreType.DMA,
            pltpu.SemaphoreType.DMA,
        ],
        compiler_params=cp,
    )
    return kern(w_packed, idx_all, x)


def _tc_body(uv_ref, o_ref):
    z = uv_ref[...]
    col = lax.broadcasted_iota(jnp.int32, z.shape, 1)
    pos = col < C_POS
    valid = col < C
    zs = jnp.where(pos, z, -z)
    ls = jnp.minimum(zs, 0.0) - jnp.log1p(jnp.exp(-jnp.abs(zs)))
    contrib = jnp.where(valid, ls, 0.0)
    o_ref[...] = -jnp.sum(contrib, axis=1)


@jax.jit
def _tc_epilogue(uv):
    blk = 2048
    return pl.pallas_call(
        _tc_body,
        grid=(B // blk,),
        in_specs=[pl.BlockSpec((blk, C_PAD), lambda i: (i, 0))],
        out_specs=pl.BlockSpec((blk,), lambda i: (i,)),
        out_shape=jax.ShapeDtypeStruct((B,), jnp.float32),
    )(uv)


def kernel(x, positive_w, negative_w, V_weight, U_weight):
    idx_all = jnp.concatenate(
        [positive_w.astype(jnp.int32), negative_w.astype(jnp.int32)], axis=1
    ).reshape(-1)
    w_rm = _tc_transpose(U_weight.T, V_weight.T)
    uv = _sc_dots(w_rm, idx_all, x.astype(jnp.int32))
    return _tc_epilogue(uv)


# speedup vs baseline: 3.6381x; 1.0136x over previous
import dataclasses

import jax
import jax.numpy as jnp
from jax import lax
from jax.experimental import pallas as pl
from jax.experimental.pallas import tpu as pltpu
from jax.experimental.pallas import tpu_sc as plsc

B = 16384
V_SIZE = 1000000
D = 64
C_POS = 20
C_NEG = 50
C = C_POS + C_NEG
C_PAD = 80
NW = 32
BPW = B // NW
NB = 8
STEPS = BPW // NB
ROWS_STEP = NB * C
GCHUNK = 112
NGC = ROWS_STEP // GCHUNK


T_BLK = 16384
T_GRID = (V_SIZE + T_BLK - 1) // T_BLK
TROWS = 2 * V_SIZE


def _transpose_body(a_ref, b_ref, o_ref):
    o_ref[...] = jnp.concatenate([a_ref[...], b_ref[...]], axis=0).T


@jax.jit
def _tc_transpose(u_t, v_t):
    out = pl.pallas_call(
        _transpose_body,
        grid=(T_GRID,),
        in_specs=[
            pl.BlockSpec((D, T_BLK), lambda i: (0, i)),
            pl.BlockSpec((D, T_BLK), lambda i: (0, i)),
        ],
        out_specs=pl.BlockSpec((T_BLK, 2 * D), lambda i: (i, 0)),
        out_shape=jax.ShapeDtypeStruct((V_SIZE, 2 * D), jnp.float32),
    )(u_t, v_t)
    return out.reshape(TROWS, D)


def _sc_body(w_hbm, idx_hbm, x_hbm, out_hbm,
             xbuf, vcs, idx_a, idx_b, rows_a, rows_b, out_a, out_b,
             sem, gsem_a, gsem_b, osem_a, osem_b):
    wid = lax.axis_index("s") * 2 + lax.axis_index("c")
    base = wid * BPW

    lane = lax.iota(jnp.int32, 16)
    masks = [lane == j for j in range(16)]

    def fire_step(s, idx_v, rows, gsem):
        b0 = base + s * NB
        pltpu.sync_copy(idx_hbm.at[pl.ds(b0 * C, ROWS_STEP)], idx_v)
        for k in range(ROWS_STEP // 16):
            iv = idx_v[pl.ds(k * 16, 16)]
            idx_v[pl.ds(k * 16, 16)] = iv + iv
        for k in range(NGC):
            pltpu.async_copy(
                w_hbm.at[idx_v.at[pl.ds(k * GCHUNK, GCHUNK)]],
                rows.at[pl.ds(k * GCHUNK, GCHUNK)], gsem)

    def wait_step(idx_v, rows, gsem):
        for k in range(NGC):
            pltpu.make_async_copy(
                w_hbm.at[idx_v.at[pl.ds(k * GCHUNK, GCHUNK)]],
                rows.at[pl.ds(k * GCHUNK, GCHUNK)], gsem).wait()

    def wait_out(out_v, osem):
        pltpu.make_async_copy(out_v, out_hbm.at[pl.ds(base, NB)],
                              osem).wait()

    def compute_step(s, rows, out_v, osem):
        @pl.loop(0, NB)
        def _row(i):
            bb = s * NB + i
            vc0 = vcs[bb, pl.ds(0, 16)]
            vc1 = vcs[bb, pl.ds(16, 16)]
            vc2 = vcs[bb, pl.ds(32, 16)]
            vc3 = vcs[bb, pl.ds(48, 16)]
            accs = [jnp.zeros((16,), jnp.float32) for _ in range(5)]
            for j in range(C):
                r = i * C + j
                t = rows[r, pl.ds(0, 16)] * vc0
                t = t + rows[r, pl.ds(16, 16)] * vc1
                t = t + rows[r, pl.ds(32, 16)] * vc2
                t = t + rows[r, pl.ds(48, 16)] * vc3
                sv = jnp.sum(t)
                g, l = divmod(j, 16)
                accs[g] = jnp.where(masks[l], sv, accs[g])
            for g in range(5):
                out_v[i, pl.ds(g * 16, 16)] = accs[g]

        pltpu.async_copy(out_v, out_hbm.at[pl.ds(base + s * NB, NB)], osem)

    pltpu.sync_copy(x_hbm.at[pl.ds(base, BPW)], xbuf)
    for k in range(BPW // 16):
        xv = xbuf[pl.ds(k * 16, 16)]
        xbuf[pl.ds(k * 16, 16)] = xv + xv + 1
    vc_copies = [
        pltpu.async_copy(
            w_hbm.at[xbuf.at[pl.ds(k * 128, 128)]],
            vcs.at[pl.ds(k * 128, 128)], sem)
        for k in range(BPW // 128)
    ]
    for cp in vc_copies:
        cp.wait()

    fire_step(0, idx_a, rows_a, gsem_a)

    @pl.loop(0, STEPS, step=2)
    def _pair(s):
        fire_step(s + 1, idx_b, rows_b, gsem_b)
        wait_step(idx_a, rows_a, gsem_a)

        @pl.when(s > 0)
        def _():
            wait_out(out_a, osem_a)

        compute_step(s, rows_a, out_a, osem_a)

        @pl.when(s + 2 < STEPS)
        def _():
            fire_step(s + 2, idx_a, rows_a, gsem_a)

        wait_step(idx_b, rows_b, gsem_b)

        @pl.when(s > 0)
        def _():
            wait_out(out_b, osem_b)

        compute_step(s + 1, rows_b, out_b, osem_b)

    wait_out(out_a, osem_a)
    wait_out(out_b, osem_b)


@jax.jit
def _sc_dots(w_packed, idx_all, x):
    mesh = plsc.VectorSubcoreMesh(core_axis_name="c", subcore_axis_name="s")
    cp = pltpu.CompilerParams()
    if "needs_layout_passes" in pltpu.CompilerParams.__dataclass_fields__:
        cp = dataclasses.replace(cp, needs_layout_passes=False)
    if "use_tc_tiling_on_sc" in pltpu.CompilerParams.__dataclass_fields__:
        cp = dataclasses.replace(cp, use_tc_tiling_on_sc=False)
    kern = pl.kernel(
        _sc_body,
        out_type=jax.ShapeDtypeStruct((B, C_PAD), jnp.float32),
        mesh=mesh,
        scratch_types=[
            pltpu.VMEM((BPW,), jnp.int32),
            pltpu.VMEM((BPW, D), jnp.float32),
            pltpu.VMEM((ROWS_STEP,), jnp.int32),
            pltpu.VMEM((ROWS_STEP,), jnp.int32),
            pltpu.VMEM((ROWS_STEP, D), jnp.float32),
            pltpu.VMEM((ROWS_STEP, D), jnp.float32),
            pltpu.VMEM((NB, C_PAD), jnp.float32),
            pltpu.VMEM((NB, C_PAD), jnp.float32),
            pltpu.SemaphoreType.DMA,
            pltpu.SemaphoreType.DMA,
            pltpu.SemaphoreType.DMA,
            pltpu.SemaphoreType.DMA,
            pltpu.SemaphoreType.DMA,
        ],
        compiler_params=cp,
    )
    return kern(w_packed, idx_all, x)


def _tc_body(uv_ref, o_ref):
    z = uv_ref[...]
    col = lax.broadcasted_iota(jnp.int32, z.shape, 1)
    pos = col < C_POS
    valid = col < C
    zs = jnp.where(pos, z, -z)
    ls = jnp.minimum(zs, 0.0) - jnp.log1p(jnp.exp(-jnp.abs(zs)))
    contrib = jnp.where(valid, ls, 0.0)
    o_ref[...] = -jnp.sum(contrib, axis=1)


@jax.jit
def _tc_epilogue(uv):
    blk = 2048
    return pl.pallas_call(
        _tc_body,
        grid=(B // blk,),
        in_specs=[pl.BlockSpec((blk, C_PAD), lambda i: (i, 0))],
        out_specs=pl.BlockSpec((blk,), lambda i: (i,)),
        out_shape=jax.ShapeDtypeStruct((B,), jnp.float32),
    )(uv)


def kernel(x, positive_w, negative_w, V_weight, U_weight):
    idx_all = jnp.concatenate(
        [positive_w.astype(jnp.int32), negative_w.astype(jnp.int32)], axis=1
    ).reshape(-1)
    w_rm = _tc_transpose(U_weight.T, V_weight.T)
    uv = _sc_dots(w_rm, idx_all, x.astype(jnp.int32))
    return _tc_epilogue(uv)


# speedup vs baseline: 3.8890x; 1.0690x over previous
import dataclasses

import jax
import jax.numpy as jnp
from jax import lax
from jax.experimental import pallas as pl
from jax.experimental.pallas import tpu as pltpu
from jax.experimental.pallas import tpu_sc as plsc

B = 16384
V_SIZE = 1000000
D = 64
C_POS = 20
C_NEG = 50
C = C_POS + C_NEG
C_PAD = 80
NW = 32
BPW = B // NW
NB = 8
STEPS = BPW // NB
ROWS_STEP = NB * C
GCHUNK = 112
NGC = ROWS_STEP // GCHUNK


T_BLK = 16384
T_GRID = (V_SIZE + T_BLK - 1) // T_BLK
TROWS = 2 * V_SIZE


def _transpose_body(a_ref, b_ref, o_ref):
    o_ref[...] = jnp.concatenate([a_ref[...], b_ref[...]], axis=0).T


@jax.jit
def _tc_transpose(u_t, v_t):
    out = pl.pallas_call(
        _transpose_body,
        grid=(T_GRID,),
        in_specs=[
            pl.BlockSpec((D, T_BLK), lambda i: (0, i)),
            pl.BlockSpec((D, T_BLK), lambda i: (0, i)),
        ],
        out_specs=pl.BlockSpec((T_BLK, 2 * D), lambda i: (i, 0)),
        out_shape=jax.ShapeDtypeStruct((V_SIZE, 2 * D), jnp.float32),
    )(u_t, v_t)
    return out.reshape(TROWS, D)


def _sc_body(w_hbm, idx_hbm, x_hbm, out_hbm,
             xbuf, vcs, idx_a, idx_b, rows_a, rows_b, out_a, out_b,
             sem, gsem_a, gsem_b, osem_a, osem_b, isem_a, isem_b):
    wid = lax.axis_index("s") * 2 + lax.axis_index("c")
    base = wid * BPW

    lane = lax.iota(jnp.int32, 16)
    masks = [lane == j for j in range(16)]

    def stage_idx(s, idx_v, isem):
        b0 = base + s * NB
        pltpu.async_copy(idx_hbm.at[pl.ds(b0 * C, ROWS_STEP)], idx_v, isem)

    def fire_step(s, idx_v, rows, gsem, isem):
        b0 = base + s * NB
        pltpu.make_async_copy(
            idx_hbm.at[pl.ds(b0 * C, ROWS_STEP)], idx_v, isem).wait()
        for k in range(ROWS_STEP // 16):
            iv = idx_v[pl.ds(k * 16, 16)]
            idx_v[pl.ds(k * 16, 16)] = iv + iv
        for k in range(NGC):
            pltpu.async_copy(
                w_hbm.at[idx_v.at[pl.ds(k * GCHUNK, GCHUNK)]],
                rows.at[pl.ds(k * GCHUNK, GCHUNK)], gsem)

    def wait_step(idx_v, rows, gsem):
        for k in range(NGC):
            pltpu.make_async_copy(
                w_hbm.at[idx_v.at[pl.ds(k * GCHUNK, GCHUNK)]],
                rows.at[pl.ds(k * GCHUNK, GCHUNK)], gsem).wait()

    def wait_out(out_v, osem):
        pltpu.make_async_copy(out_v, out_hbm.at[pl.ds(base, NB)],
                              osem).wait()

    def compute_step(s, rows, out_v, osem):
        @pl.loop(0, NB)
        def _row(i):
            bb = s * NB + i
            vc0 = vcs[bb, pl.ds(0, 16)]
            vc1 = vcs[bb, pl.ds(16, 16)]
            vc2 = vcs[bb, pl.ds(32, 16)]
            vc3 = vcs[bb, pl.ds(48, 16)]
            accs = [jnp.zeros((16,), jnp.float32) for _ in range(5)]
            for j in range(C):
                r = i * C + j
                t = rows[r, pl.ds(0, 16)] * vc0
                t = t + rows[r, pl.ds(16, 16)] * vc1
                t = t + rows[r, pl.ds(32, 16)] * vc2
                t = t + rows[r, pl.ds(48, 16)] * vc3
                sv = jnp.sum(t)
                g, l = divmod(j, 16)
                accs[g] = jnp.where(masks[l], sv, accs[g])
            for g in range(5):
                out_v[i, pl.ds(g * 16, 16)] = accs[g]

        pltpu.async_copy(out_v, out_hbm.at[pl.ds(base + s * NB, NB)], osem)

    pltpu.sync_copy(x_hbm.at[pl.ds(base, BPW)], xbuf)
    for k in range(BPW // 16):
        xv = xbuf[pl.ds(k * 16, 16)]
        xbuf[pl.ds(k * 16, 16)] = xv + xv + 1
    vc_copies = [
        pltpu.async_copy(
            w_hbm.at[xbuf.at[pl.ds(k * 128, 128)]],
            vcs.at[pl.ds(k * 128, 128)], sem)
        for k in range(BPW // 128)
    ]
    for cp in vc_copies:
        cp.wait()

    stage_idx(0, idx_a, isem_a)
    stage_idx(1, idx_b, isem_b)
    fire_step(0, idx_a, rows_a, gsem_a, isem_a)

    @pl.loop(0, STEPS, step=2)
    def _pair(s):
        fire_step(s + 1, idx_b, rows_b, gsem_b, isem_b)
        wait_step(idx_a, rows_a, gsem_a)

        @pl.when(s + 2 < STEPS)
        def _():
            stage_idx(s + 2, idx_a, isem_a)

        @pl.when(s > 0)
        def _():
            wait_out(out_a, osem_a)

        compute_step(s, rows_a, out_a, osem_a)

        @pl.when(s + 2 < STEPS)
        def _():
            fire_step(s + 2, idx_a, rows_a, gsem_a, isem_a)

        wait_step(idx_b, rows_b, gsem_b)

        @pl.when(s + 3 < STEPS)
        def _():
            stage_idx(s + 3, idx_b, isem_b)

        @pl.when(s > 0)
        def _():
            wait_out(out_b, osem_b)

        compute_step(s + 1, rows_b, out_b, osem_b)

    wait_out(out_a, osem_a)
    wait_out(out_b, osem_b)


@jax.jit
def _sc_dots(w_packed, idx_all, x):
    mesh = plsc.VectorSubcoreMesh(core_axis_name="c", subcore_axis_name="s")
    cp = pltpu.CompilerParams()
    if "needs_layout_passes" in pltpu.CompilerParams.__dataclass_fields__:
        cp = dataclasses.replace(cp, needs_layout_passes=False)
    if "use_tc_tiling_on_sc" in pltpu.CompilerParams.__dataclass_fields__:
        cp = dataclasses.replace(cp, use_tc_tiling_on_sc=False)
    kern = pl.kernel(
        _sc_body,
        out_type=jax.ShapeDtypeStruct((B, C_PAD), jnp.float32),
        mesh=mesh,
        scratch_types=[
            pltpu.VMEM((BPW,), jnp.int32),
            pltpu.VMEM((BPW, D), jnp.float32),
            pltpu.VMEM((ROWS_STEP,), jnp.int32),
            pltpu.VMEM((ROWS_STEP,), jnp.int32),
            pltpu.VMEM((ROWS_STEP, D), jnp.float32),
            pltpu.VMEM((ROWS_STEP, D), jnp.float32),
            pltpu.VMEM((NB, C_PAD), jnp.float32),
            pltpu.VMEM((NB, C_PAD), jnp.float32),
            pltpu.SemaphoreType.DMA,
            pltpu.SemaphoreType.DMA,
            pltpu.SemaphoreType.DMA,
            pltpu.SemaphoreType.DMA,
            pltpu.SemaphoreType.DMA,
            pltpu.SemaphoreType.DMA,
            pltpu.SemaphoreType.DMA,
        ],
        compiler_params=cp,
    )
    return kern(w_packed, idx_all, x)


def _tc_body(uv_ref, o_ref):
    z = uv_ref[...]
    col = lax.broadcasted_iota(jnp.int32, z.shape, 1)
    pos = col < C_POS
    valid = col < C
    zs = jnp.where(pos, z, -z)
    ls = jnp.minimum(zs, 0.0) - jnp.log1p(jnp.exp(-jnp.abs(zs)))
    contrib = jnp.where(valid, ls, 0.0)
    o_ref[...] = -jnp.sum(contrib, axis=1)


@jax.jit
def _tc_epilogue(uv):
    blk = 2048
    return pl.pallas_call(
        _tc_body,
        grid=(B // blk,),
        in_specs=[pl.BlockSpec((blk, C_PAD), lambda i: (i, 0))],
        out_specs=pl.BlockSpec((blk,), lambda i: (i,)),
        out_shape=jax.ShapeDtypeStruct((B,), jnp.float32),
    )(uv)


def kernel(x, positive_w, negative_w, V_weight, U_weight):
    idx_all = jnp.concatenate(
        [positive_w.astype(jnp.int32), negative_w.astype(jnp.int32)], axis=1
    ).reshape(-1)
    w_rm = _tc_transpose(U_weight.T, V_weight.T)
    uv = _sc_dots(w_rm, idx_all, x.astype(jnp.int32))
    return _tc_epilogue(uv)
